# baseline (speedup 1.0000x reference)
import jax
import jax.numpy as jnp
from jax.experimental import pallas as pl


def _norm_body(v_ref, o_ref):
    v = v_ref[...]
    o_ref[...] = jnp.sqrt(jnp.sum(v * v, axis=-1))


def kernel(edge_index, edge_vec):
    E = edge_vec.shape[0]
    B = 25600
    w = pl.pallas_call(
        _norm_body,
        grid=(E // B,),
        in_specs=[pl.BlockSpec((B, 3), lambda i: (i, 0))],
        out_specs=pl.BlockSpec((B,), lambda i: (i,)),
        out_shape=jax.ShapeDtypeStruct((E,), jnp.float32),
    )(edge_vec)
    return (edge_index, w, edge_vec)


# speedup vs baseline: 9.9053x; 9.9053x over previous
import functools

import jax
import jax.numpy as jnp
from jax import lax
from jax.experimental import pallas as pl
from jax.experimental.pallas import tpu as pltpu
from jax.experimental.pallas import tpu_sc as plsc

_NC = 2
_NS = 16
_NW = _NC * _NS
_CB = 5120


def _sqrt16(s):
    i = lax.bitcast_convert_type(s, jnp.int32)
    i = 0x5F3759DF - lax.shift_right_arithmetic(i, 1)
    y = lax.bitcast_convert_type(i, jnp.float32)
    y = y * (1.5 - 0.5 * s * y * y)
    y = y * (1.5 - 0.5 * s * y * y)
    y = y * (1.5 - 0.5 * s * y * y)
    return s * y


def _make_norm_kernel(E):
    nchunk = E // _CB
    mesh = plsc.VectorSubcoreMesh(core_axis_name="c", subcore_axis_name="s")

    @functools.partial(
        pl.kernel,
        mesh=mesh,
        out_type=jax.ShapeDtypeStruct((E,), jnp.float32),
        scratch_types=[
            pltpu.VMEM((3, _CB), jnp.float32),
            pltpu.VMEM((_CB,), jnp.float32),
        ],
    )
    def norm_k(evt_hbm, out_hbm, vb, ob):
        wid = lax.axis_index("s") * _NC + lax.axis_index("c")
        nk = (nchunk - wid + _NW - 1) // _NW

        def chunk(i, carry):
            off = (wid + i * _NW) * _CB
            pltpu.sync_copy(evt_hbm.at[:, pl.ds(off, _CB)], vb)

            def grp(g, c):
                x = vb[0, pl.ds(g * 16, 16)]
                y = vb[1, pl.ds(g * 16, 16)]
                z = vb[2, pl.ds(g * 16, 16)]
                ob[pl.ds(g * 16, 16)] = _sqrt16(x * x + y * y + z * z)
                return c

            lax.fori_loop(0, _CB // 16, grp, 0)
            pltpu.sync_copy(ob, out_hbm.at[pl.ds(off, _CB)])
            return carry

        lax.fori_loop(0, nk, chunk, 0)

    return norm_k


def kernel(edge_index, edge_vec):
    E = edge_vec.shape[0]
    w = _make_norm_kernel(E)(edge_vec.T)
    return (edge_index, w, edge_vec)


# speedup vs baseline: 13.0135x; 1.3138x over previous
import functools

import jax
import jax.numpy as jnp
from jax import lax
from jax.experimental import pallas as pl
from jax.experimental.pallas import tpu as pltpu
from jax.experimental.pallas import tpu_sc as plsc

_NC = 2
_NS = 16
_NW = _NC * _NS
_CB = 5120


def _sqrt16(s):
    i = lax.bitcast_convert_type(s, jnp.int32)
    i = 0x5F3759DF - lax.shift_right_arithmetic(i, 1)
    y = lax.bitcast_convert_type(i, jnp.float32)
    y = y * (1.5 - 0.5 * s * y * y)
    y = y * (1.5 - 0.5 * s * y * y)
    return s * y


def _make_norm_kernel(E):
    nchunk = E // _CB
    mesh = plsc.VectorSubcoreMesh(core_axis_name="c", subcore_axis_name="s")

    @functools.partial(
        pl.kernel,
        mesh=mesh,
        out_type=jax.ShapeDtypeStruct((E,), jnp.float32),
        scratch_types=[
            pltpu.VMEM((3, _CB), jnp.float32),
            pltpu.VMEM((3, _CB), jnp.float32),
            pltpu.VMEM((_CB,), jnp.float32),
            pltpu.VMEM((_CB,), jnp.float32),
            pltpu.SemaphoreType.DMA,
            pltpu.SemaphoreType.DMA,
            pltpu.SemaphoreType.DMA,
            pltpu.SemaphoreType.DMA,
        ],
    )
    def norm_k(evt_hbm, out_hbm, vb0, vb1, ob0, ob1, si0, si1, so0, so1):
        wid = lax.axis_index("s") * _NC + lax.axis_index("c")
        nk = (nchunk - wid + _NW - 1) // _NW
        vbs, obs = (vb0, vb1), (ob0, ob1)
        sis, sos = (si0, si1), (so0, so1)

        def in_copy(i, b):
            off = (wid + i * _NW) * _CB
            return pltpu.make_async_copy(
                evt_hbm.at[:, pl.ds(off, _CB)], vbs[b], sis[b])

        def out_copy(i, b):
            off = (wid + i * _NW) * _CB
            return pltpu.make_async_copy(
                obs[b], out_hbm.at[pl.ds(off, _CB)], sos[b])

        @pl.when(nk > 0)
        def _():
            in_copy(0, 0).start()

        @pl.when(nk > 1)
        def _():
            in_copy(1, 1).start()

        def run_chunk(i, b):
            vb, ob = vbs[b], obs[b]

            @pl.when(i >= 2)
            def _():
                out_copy(i - 2, b).wait()

            in_copy(i, b).wait()

            def grp(g, c):
                x = vb[0, pl.ds(g * 16, 16)]
                y = vb[1, pl.ds(g * 16, 16)]
                z = vb[2, pl.ds(g * 16, 16)]
                ob[pl.ds(g * 16, 16)] = _sqrt16(x * x + y * y + z * z)
                return c

            lax.fori_loop(0, _CB // 16, grp, 0, unroll=8)
            out_copy(i, b).start()

            @pl.when(i + 2 < nk)
            def _():
                in_copy(i + 2, b).start()

        def pair(p, c):
            for b in range(2):
                i = 2 * p + b

                @pl.when(i < nk)
                def _():
                    run_chunk(i, b)

            return c

        lax.fori_loop(0, (nchunk // _NW + 2) // 2, pair, 0)

        for b in range(2):
            i_b = nk - 1 - lax.rem(nk - 1 - b, 2)

            @pl.when((i_b >= 0) & (nk > b))
            def _(i_b=i_b, b=b):
                out_copy(i_b, b).wait()

    return norm_k


def kernel(edge_index, edge_vec):
    E = edge_vec.shape[0]
    w = _make_norm_kernel(E)(edge_vec.T)
    return (edge_index, w, edge_vec)


# speedup vs baseline: 17.1982x; 1.3216x over previous
import functools

import jax
import jax.numpy as jnp
from jax import lax
from jax.experimental import pallas as pl
from jax.experimental.pallas import tpu as pltpu
from jax.experimental.pallas import tpu_sc as plsc

_NC = 2
_NS = 16
_NW = _NC * _NS
_CB = 5120


def _sqrt16(s):
    i = lax.bitcast_convert_type(s, jnp.int32)
    i = 0x5F3759DF - lax.shift_right_arithmetic(i, 1)
    y = lax.bitcast_convert_type(i, jnp.float32)
    y = y * (1.5 - 0.5 * s * y * y)
    y = y * (1.5 - 0.5 * s * y * y)
    return s * y


def _make_norm_kernel(E):
    nchunk = E // _CB
    mesh = plsc.VectorSubcoreMesh(core_axis_name="c", subcore_axis_name="s")

    @functools.partial(
        pl.kernel,
        mesh=mesh,
        out_type=[
            jax.ShapeDtypeStruct((E,), jnp.float32),
            jax.ShapeDtypeStruct((2, E), jnp.int32),
            jax.ShapeDtypeStruct((3, E), jnp.float32),
        ],
        scratch_types=[
            pltpu.VMEM((3, _CB), jnp.float32),
            pltpu.VMEM((3, _CB), jnp.float32),
            pltpu.VMEM((2, _CB), jnp.int32),
            pltpu.VMEM((2, _CB), jnp.int32),
            pltpu.VMEM((_CB,), jnp.float32),
            pltpu.VMEM((_CB,), jnp.float32),
        ] + [pltpu.SemaphoreType.DMA] * 12,
    )
    def norm_k(evt_hbm, eidx_hbm, w_hbm, eidxc_hbm, evtc_hbm,
               vb0, vb1, ib0, ib1, ob0, ob1,
               siv0, siv1, sii0, sii1, sov0, sov1, soi0, soi1,
               sow0, sow1, sxx0, sxx1):
        wid = lax.axis_index("s") * _NC + lax.axis_index("c")
        nk = (nchunk - wid + _NW - 1) // _NW
        vbs, ibs, obs = (vb0, vb1), (ib0, ib1), (ob0, ob1)
        sivs, siis = (siv0, siv1), (sii0, sii1)
        sovs, sois, sows = (sov0, sov1), (soi0, soi1), (sow0, sow1)

        def off_of(i):
            return (wid + i * _NW) * _CB

        def in_v(i, b):
            return pltpu.make_async_copy(
                evt_hbm.at[:, pl.ds(off_of(i), _CB)], vbs[b], sivs[b])

        def in_i(i, b):
            return pltpu.make_async_copy(
                eidx_hbm.at[:, pl.ds(off_of(i), _CB)], ibs[b], siis[b])

        def out_v(i, b):
            return pltpu.make_async_copy(
                vbs[b], evtc_hbm.at[:, pl.ds(off_of(i), _CB)], sovs[b])

        def out_i(i, b):
            return pltpu.make_async_copy(
                ibs[b], eidxc_hbm.at[:, pl.ds(off_of(i), _CB)], sois[b])

        def out_w(i, b):
            return pltpu.make_async_copy(
                obs[b], w_hbm.at[pl.ds(off_of(i), _CB)], sows[b])

        for b in range(2):
            @pl.when(nk > b)
            def _(b=b):
                in_v(b, b).start()
                in_i(b, b).start()

        def run_chunk(i, b):
            vb, ob = vbs[b], obs[b]
            in_v(i, b).wait()
            in_i(i, b).wait()
            out_v(i, b).start()
            out_i(i, b).start()

            @pl.when(i >= 2)
            def _():
                out_w(i - 2, b).wait()

            def grp(g, c):
                x = vb[0, pl.ds(g * 16, 16)]
                y = vb[1, pl.ds(g * 16, 16)]
                z = vb[2, pl.ds(g * 16, 16)]
                ob[pl.ds(g * 16, 16)] = _sqrt16(x * x + y * y + z * z)
                return c

            lax.fori_loop(0, _CB // 16, grp, 0, unroll=8)
            out_w(i, b).start()

            @pl.when(i + 2 < nk)
            def _():
                out_v(i, b).wait()
                out_i(i, b).wait()
                in_v(i + 2, b).start()
                in_i(i + 2, b).start()

        def pair(p, c):
            for b in range(2):
                i = 2 * p + b

                @pl.when(i < nk)
                def _(i=i, b=b):
                    run_chunk(i, b)

            return c

        lax.fori_loop(0, ((nchunk + _NW - 1) // _NW + 1) // 2, pair, 0)

        for b in range(2):
            i_b = nk - 1 - lax.rem(nk - 1 - b, 2)

            @pl.when((i_b >= 0) & (nk > b))
            def _(i_b=i_b, b=b):
                out_w(i_b, b).wait()
                out_v(i_b, b).wait()
                out_i(i_b, b).wait()

    return norm_k


def kernel(edge_index, edge_vec):
    E = edge_vec.shape[0]
    w, eidx_c, evt_c = _make_norm_kernel(E)(edge_vec.T, edge_index)
    return (eidx_c, w, evt_c.T)


# speedup vs baseline: 31.4934x; 1.8312x over previous
import functools

import jax
import jax.numpy as jnp
from jax import lax
from jax.experimental import pallas as pl
from jax.experimental.pallas import tpu as pltpu
from jax.experimental.pallas import tpu_sc as plsc

_NC = 2
_NS = 16
_NW = _NC * _NS
_CB = 5120


def _sqrt16(s):
    i = lax.bitcast_convert_type(s, jnp.int32)
    i = 0x5F3759DF - lax.shift_right_arithmetic(i, 1)
    y = lax.bitcast_convert_type(i, jnp.float32)
    y = y * (1.5 - 0.5 * s * y * y)
    y = y * (1.5 - 0.5 * s * y * y)
    return s * y


def _make_norm_kernel(E):
    nchunk = E // _CB
    mesh = plsc.VectorSubcoreMesh(core_axis_name="c", subcore_axis_name="s")

    @functools.partial(
        pl.kernel,
        mesh=mesh,
        out_type=[
            jax.ShapeDtypeStruct((E,), jnp.float32),
            jax.ShapeDtypeStruct((2, E), jnp.int32),
            jax.ShapeDtypeStruct((3, E), jnp.float32),
        ],
        scratch_types=[
            pltpu.VMEM((3, _CB), jnp.float32),
            pltpu.VMEM((3, _CB), jnp.float32),
            pltpu.VMEM((2, _CB), jnp.int32),
            pltpu.VMEM((2, _CB), jnp.int32),
            pltpu.VMEM((_CB,), jnp.float32),
            pltpu.VMEM((_CB,), jnp.float32),
        ] + [pltpu.SemaphoreType.DMA] * 12,
    )
    def norm_k(evt_hbm, eidx_hbm, w_hbm, eidxc_hbm, evtc_hbm,
               vb0, vb1, ib0, ib1, ob0, ob1,
               siv0, siv1, sii0, sii1, sov0, sov1, soi0, soi1,
               sow0, sow1, sxx0, sxx1):
        wid = lax.axis_index("s") * _NC + lax.axis_index("c")
        nk = (nchunk - wid + _NW - 1) // _NW
        vbs, ibs, obs = (vb0, vb1), (ib0, ib1), (ob0, ob1)
        sivs, siis = (siv0, siv1), (sii0, sii1)
        sovs, sois, sows = (sov0, sov1), (soi0, soi1), (sow0, sow1)

        def off_of(i):
            return (wid + i * _NW) * _CB

        def in_v(i, b):
            return pltpu.make_async_copy(
                evt_hbm.at[:, pl.ds(off_of(i), _CB)], vbs[b], sivs[b])

        def in_i(i, b):
            return pltpu.make_async_copy(
                eidx_hbm.at[:, pl.ds(off_of(i), _CB)], ibs[b], siis[b])

        def out_v(i, b):
            return pltpu.make_async_copy(
                vbs[b], evtc_hbm.at[:, pl.ds(off_of(i), _CB)], sovs[b])

        def out_i(i, b):
            return pltpu.make_async_copy(
                ibs[b], eidxc_hbm.at[:, pl.ds(off_of(i), _CB)], sois[b])

        def out_w(i, b):
            return pltpu.make_async_copy(
                obs[b], w_hbm.at[pl.ds(off_of(i), _CB)], sows[b])

        for b in range(2):
            @pl.when(nk > b)
            def _(b=b):
                in_v(b, b).start()
                in_i(b, b).start()

        def run_chunk(i, b):
            vb, ob = vbs[b], obs[b]
            in_v(i, b).wait()
            in_i(i, b).wait()
            out_v(i, b).start()
            out_i(i, b).start()

            @pl.when(i >= 2)
            def _():
                out_w(i - 2, b).wait()

            @plsc.parallel_loop(0, _CB // 16, unroll=8)
            def _(g):
                x = vb[0, pl.ds(g * 16, 16)]
                y = vb[1, pl.ds(g * 16, 16)]
                z = vb[2, pl.ds(g * 16, 16)]
                ob[pl.ds(g * 16, 16)] = _sqrt16(x * x + y * y + z * z)
            out_w(i, b).start()

            @pl.when(i + 2 < nk)
            def _():
                out_v(i, b).wait()
                out_i(i, b).wait()
                in_v(i + 2, b).start()
                in_i(i + 2, b).start()

        def pair(p, c):
            for b in range(2):
                i = 2 * p + b

                @pl.when(i < nk)
                def _(i=i, b=b):
                    run_chunk(i, b)

            return c

        lax.fori_loop(0, ((nchunk + _NW - 1) // _NW + 1) // 2, pair, 0)

        for b in range(2):
            i_b = nk - 1 - lax.rem(nk - 1 - b, 2)

            @pl.when((i_b >= 0) & (nk > b))
            def _(i_b=i_b, b=b):
                out_w(i_b, b).wait()
                out_v(i_b, b).wait()
                out_i(i_b, b).wait()

    return norm_k


def kernel(edge_index, edge_vec):
    E = edge_vec.shape[0]
    w, eidx_c, evt_c = _make_norm_kernel(E)(edge_vec.T, edge_index)
    return (eidx_c, w, evt_c.T)


# speedup vs baseline: 31.6888x; 1.0062x over previous
import functools

import jax
import jax.numpy as jnp
from jax import lax
from jax.experimental import pallas as pl
from jax.experimental.pallas import tpu as pltpu
from jax.experimental.pallas import tpu_sc as plsc

_NC = 2
_NS = 16
_NW = _NC * _NS
_CB = 5120
_NB = 3


def _sqrt16(s):
    i = lax.bitcast_convert_type(s, jnp.int32)
    i = 0x5F3759DF - lax.shift_right_arithmetic(i, 1)
    y = lax.bitcast_convert_type(i, jnp.float32)
    y = y * (1.5 - 0.5 * s * y * y)
    y = y * (1.5 - 0.5 * s * y * y)
    return s * y


def _make_norm_kernel(E):
    nchunk = E // _CB
    mesh = plsc.VectorSubcoreMesh(core_axis_name="c", subcore_axis_name="s")

    @functools.partial(
        pl.kernel,
        mesh=mesh,
        out_type=[
            jax.ShapeDtypeStruct((E,), jnp.float32),
            jax.ShapeDtypeStruct((2, E), jnp.int32),
            jax.ShapeDtypeStruct((3, E), jnp.float32),
        ],
        scratch_types=(
            [pltpu.VMEM((3, _CB), jnp.float32)] * _NB
            + [pltpu.VMEM((2, _CB), jnp.int32)] * _NB
            + [pltpu.VMEM((_CB,), jnp.float32)] * _NB
            + [pltpu.SemaphoreType.DMA] * (5 * _NB)
        ),
    )
    def norm_k(evt_hbm, eidx_hbm, w_hbm, eidxc_hbm, evtc_hbm, *bufs):
        vbs = bufs[0:_NB]
        ibs = bufs[_NB:2 * _NB]
        obs = bufs[2 * _NB:3 * _NB]
        sems = bufs[3 * _NB:]
        sivs, siis = sems[0:_NB], sems[_NB:2 * _NB]
        sovs, sois = sems[2 * _NB:3 * _NB], sems[3 * _NB:4 * _NB]
        sows = sems[4 * _NB:5 * _NB]

        wid = lax.axis_index("s") * _NC + lax.axis_index("c")
        nk = (nchunk - wid + _NW - 1) // _NW

        def off_of(i):
            return (wid + i * _NW) * _CB

        def in_v(i, b):
            return pltpu.make_async_copy(
                evt_hbm.at[:, pl.ds(off_of(i), _CB)], vbs[b], sivs[b])

        def in_i(i, b):
            return pltpu.make_async_copy(
                eidx_hbm.at[:, pl.ds(off_of(i), _CB)], ibs[b], siis[b])

        def out_v(i, b):
            return pltpu.make_async_copy(
                vbs[b], evtc_hbm.at[:, pl.ds(off_of(i), _CB)], sovs[b])

        def out_i(i, b):
            return pltpu.make_async_copy(
                ibs[b], eidxc_hbm.at[:, pl.ds(off_of(i), _CB)], sois[b])

        def out_w(i, b):
            return pltpu.make_async_copy(
                obs[b], w_hbm.at[pl.ds(off_of(i), _CB)], sows[b])

        for b in range(2):
            @pl.when(nk > b)
            def _(b=b):
                in_v(b, b).start()
                in_i(b, b).start()

        def run_chunk(i, b):
            bn = (b + 2) % _NB

            @pl.when(i + 2 < nk)
            def _():
                @pl.when(i >= 1)
                def _():
                    out_v(i - 1, bn).wait()
                    out_i(i - 1, bn).wait()

                in_v(i + 2, bn).start()
                in_i(i + 2, bn).start()

            vb, ob = vbs[b], obs[b]
            in_v(i, b).wait()
            in_i(i, b).wait()
            out_v(i, b).start()
            out_i(i, b).start()

            @pl.when(i >= _NB)
            def _():
                out_w(i - _NB, b).wait()

            @plsc.parallel_loop(0, _CB // 16, unroll=8)
            def _(g):
                x = vb[0, pl.ds(g * 16, 16)]
                y = vb[1, pl.ds(g * 16, 16)]
                z = vb[2, pl.ds(g * 16, 16)]
                ob[pl.ds(g * 16, 16)] = _sqrt16(x * x + y * y + z * z)

            out_w(i, b).start()

        def triple(p, c):
            for b in range(_NB):
                i = _NB * p + b

                @pl.when(i < nk)
                def _(i=i, b=b):
                    run_chunk(i, b)

            return c

        nk_max = (nchunk + _NW - 1) // _NW
        lax.fori_loop(0, (nk_max + _NB - 1) // _NB, triple, 0)

        for b in range(_NB):
            i_b = nk - 1 - lax.rem(nk - 1 - b, _NB)

            @pl.when(nk > b)
            def _(i_b=i_b, b=b):
                out_w(i_b, b).wait()
                out_v(i_b, b).wait()
                out_i(i_b, b).wait()

    return norm_k


def kernel(edge_index, edge_vec):
    E = edge_vec.shape[0]
    w, eidx_c, evt_c = _make_norm_kernel(E)(edge_vec.T, edge_index)
    return (eidx_c, w, evt_c.T)
